# baseline (speedup 1.0000x reference)
import math

import jax
import jax.numpy as jnp
from jax.experimental import pallas as pl
from jax.experimental.pallas import tpu as pltpu

_S = 64.0
_M2 = 0.5
_COS_M = math.cos(_M2)
_SIN_M = math.sin(_M2)
_THETA = math.cos(math.pi - _M2)
_SINMM = math.sin(math.pi - _M2) * _M2

_BLOCK_N = 2048


def _body(labels_ref, x_ref, o_ref):
    j = pl.program_id(0)
    t = x_ref[...]
    rows, cols = t.shape
    col = jax.lax.broadcasted_iota(jnp.int32, (rows, cols), 1) + j * cols
    mask = col == labels_ref[...]
    cos_theta_m = t * _COS_M - jnp.sqrt(1.0 - t * t) * _SIN_M
    f = jnp.where(t > _THETA, cos_theta_m, t - _SINMM)
    o_ref[...] = jnp.where(mask, f, t) * _S


def kernel(logits, labels):
    b, n = logits.shape
    grid = (pl.cdiv(n, _BLOCK_N),)
    labels2d = labels.reshape(b, 1)
    return pl.pallas_call(
        _body,
        grid=grid,
        in_specs=[
            pl.BlockSpec((b, 1), lambda j: (0, 0)),
            pl.BlockSpec((b, _BLOCK_N), lambda j: (0, j)),
        ],
        out_specs=pl.BlockSpec((b, _BLOCK_N), lambda j: (0, j)),
        out_shape=jax.ShapeDtypeStruct((b, n), jnp.float32),
        compiler_params=pltpu.CompilerParams(
            dimension_semantics=("arbitrary",),
        ),
    )(labels2d, logits)
